# baseline (speedup 1.0000x reference)
import functools

import jax
import jax.numpy as jnp
from jax import lax
from jax.experimental import pallas as pl
from jax.experimental.pallas import tpu as pltpu
from jax.experimental.pallas import tpu_sc as plsc

VOCAB = 1000000
EMBED = 64
BATCH = 16384

_INFO = plsc.get_sparse_core_info()
_NC = _INFO.num_cores
_NS = _INFO.num_subcores
_NW = _NC * _NS

_SLABW = 128
_NSLAB = VOCAB // _SLABW
_TAIL_BASE = _NSLAB * _SLABW
_TAILW = VOCAB - _TAIL_BASE
_SPW = _NSLAB // _NW
_SREM = _NSLAB % _NW
_LCAP = 1024
_STAGE = 128
_OUTROWS = BATCH + _NW
_IDXCH = 2048


@functools.partial(
    pl.kernel,
    mesh=plsc.VectorSubcoreMesh(core_axis_name="c", subcore_axis_name="s"),
    out_type=(
        jax.ShapeDtypeStruct((_OUTROWS, 2 * EMBED), jnp.float32),
        jax.ShapeDtypeStruct((_OUTROWS, 2 * EMBED), jnp.float32),
    ),
    scratch_types=[
        pltpu.VMEM((8, EMBED, _SLABW), jnp.float32),
        pltpu.VMEM((EMBED, _TAILW), jnp.float32),
        pltpu.VMEM((_IDXCH,), jnp.int32),
        pltpu.VMEM((_LCAP + 16,), jnp.int32),
        pltpu.VMEM((_LCAP + 16,), jnp.int32),
        pltpu.VMEM((_LCAP + 16,), jnp.int32),
        pltpu.VMEM((_LCAP + 16,), jnp.int32),
        pltpu.VMEM((1, _STAGE), jnp.int32),
        pltpu.VMEM((_STAGE, 2 * EMBED), jnp.float32),
        pltpu.SemaphoreType.DMA,
        pltpu.SemaphoreType.DMA,
        pltpu.SemaphoreType.DMA,
        pltpu.SemaphoreType.DMA,
        pltpu.SemaphoreType.DMA,
        pltpu.SemaphoreType.DMA,
        pltpu.SemaphoreType.DMA,
        pltpu.SemaphoreType.DMA,
    ],
    compiler_params=pltpu.CompilerParams(needs_layout_passes=False),
)
def _skipgram_stream(center_hbm, context_hbm, inT_hbm, outT_hbm,
                     center_out_hbm, context_out_hbm,
                     slab_v, tail_v, idxs_v, mi_v, mk_v, scol_v, sk_v,
                     k2d_v, stage_v, sem0, sem1, sem2, sem3, sem4, sem5, sem6, sem7):
    wid = lax.axis_index("s") * _NC + lax.axis_index("c")
    lanes = lax.iota(jnp.int32, 16)
    zeros16 = lax.full((16,), 0, jnp.int32)
    sems = (sem0, sem1, sem2, sem3, sem4, sem5, sem6, sem7)

    s_lo = wid * _SPW + jnp.minimum(wid, _SREM)
    nslab = _SPW + jnp.where(wid < _SREM, 1, 0)
    lo = pl.multiple_of(s_lo * _SLABW, 128)
    is_last = wid == (_NW - 1)
    hi = jnp.where(is_last, VOCAB, lo + nslab * _SLABW)
    dummy_row = BATCH + wid

    def reset_k2d():
        row = k2d_v.at[0]
        for t in range(_STAGE // 16):
            row[pl.ds(t * 16, 16)] = zeros16 + dummy_row

    def flush(out_hbm):
        pltpu.sync_copy(stage_v, out_hbm.at[k2d_v.at[0]])
        reset_k2d()

    def fire(tblT_hbm, si, b):
        base = pl.multiple_of((s_lo + si) * _SLABW, 128)
        return pltpu.async_copy(
            tblT_hbm.at[:, pl.ds(base, _SLABW)], slab_v.at[b], sems[b])

    def process_slab(slabref, base, width, listn, st, out_hbm):
        def collect(v, sp):
            mi = mi_v[pl.ds(v * 16, 16)]
            kk = mk_v[pl.ds(v * 16, 16)]
            act = (v * 16 + lanes) < listn
            m = act & (mi >= base) & (mi < base + width)
            plsc.store_compressed(scol_v.at[pl.ds(sp, 16)], mi - base, mask=m)
            plsc.store_compressed(sk_v.at[pl.ds(sp, 16)], kk, mask=m)
            return sp + plsc.all_reduce_population_count(m)[0]

        sp = lax.fori_loop(0, (listn + 15) // 16, collect, 0)

        def extract(q, st):
            colv = scol_v[pl.ds(q * 16, 16)]
            kvv = sk_v[pl.ds(q * 16, 16)]
            mm = (q * 16 + lanes) < sp
            plsc.store_scatter(k2d_v.at[0], [st + lanes], kvv, mask=mm)

            def col_body(c, carry):
                val = plsc.load_gather(slabref, [zeros16 + c, colv], mask=mm)
                plsc.store_scatter(stage_v, [st + lanes, zeros16 + c], val,
                                  mask=mm)
                return carry

            lax.fori_loop(0, EMBED, col_body, 0)
            st = st + plsc.all_reduce_population_count(mm)[0]

            def do_flush():
                flush(out_hbm)
                return 0

            return lax.cond(st > _STAGE - 16, do_flush, lambda: st)

        return lax.fori_loop(0, (sp + 15) // 16, extract, st)

    def table_pass(idx_hbm, tblT_hbm, out_hbm):
        def round_body(carry):
            r, _ = carry
            reset_k2d()
            for b in range(8):
                fire(tblT_hbm, b, b)

            rbase = r * _LCAP

            def scan_chunk(ch_base, pos):
                def scanb(b, pos):
                    iv = idxs_v[pl.ds(b * 16, 16)]
                    kv = lanes + (ch_base + b * 16)
                    m = (iv >= lo) & (iv < hi)
                    cnt = plsc.all_reduce_population_count(m)[0]
                    lpos = pos - rbase

                    def fast_append():
                        at = jnp.maximum(lpos, 0)
                        plsc.store_compressed(mi_v.at[pl.ds(at, 16)], iv,
                                              mask=m)
                        plsc.store_compressed(mk_v.at[pl.ds(at, 16)], kv,
                                              mask=m)

                    def ranked_append():
                        rank = lpos + plsc.cumsum(
                            jnp.where(m, 1, 0).astype(jnp.int32)) - 1
                        acc = m & (rank >= 0) & (rank < _LCAP)
                        at = jnp.maximum(lpos, 0)
                        plsc.store_compressed(mi_v.at[pl.ds(at, 16)], iv,
                                              mask=acc)
                        plsc.store_compressed(mk_v.at[pl.ds(at, 16)], kv,
                                              mask=acc)

                    lax.cond((lpos >= 0) & (lpos <= _LCAP - 16),
                             fast_append, ranked_append)
                    return pos + cnt

                return lax.fori_loop(0, _IDXCH // 16, scanb, pos)

            pos = 0
            for ch in range(BATCH // _IDXCH):
                pltpu.sync_copy(idx_hbm.at[pl.ds(ch * _IDXCH, _IDXCH)],
                                idxs_v)
                pos = scan_chunk(ch * _IDXCH, pos)
            total = pos
            listn = jnp.clip(total - rbase, 0, _LCAP)

            def pair_body(g, st):
                for b in range(8):
                    si = g * 8 + b

                    def do_slab(st=st, si=si, b=b):
                        pltpu.make_async_copy(
                            tblT_hbm.at[:, pl.ds(0, _SLABW)],
                            slab_v.at[b], sems[b]).wait()
                        base = lo + si * _SLABW
                        st2 = process_slab(slab_v.at[b], base, _SLABW,
                                           listn, st, out_hbm)

                        @pl.when(si + 8 < nslab)
                        def _():
                            fire(tblT_hbm, si + 8, b)

                        return st2

                    st = lax.cond(si < nslab, do_slab, lambda st=st: st)
                return st

            st = lax.fori_loop(0, (_SPW + 1 + 8) // 8, pair_body, 0)

            def do_tail(st=st):
                pltpu.sync_copy(
                    tblT_hbm.at[:, pl.ds(_TAIL_BASE, _TAILW)], tail_v)
                return process_slab(tail_v, _TAIL_BASE, _TAILW, listn, st,
                                    out_hbm)

            st = lax.cond(is_last, do_tail, lambda: st)

            @pl.when(st > 0)
            def _():
                flush(out_hbm)

            return r + 1, total

        lax.while_loop(
            lambda c: (c[0] == 0) | (c[0] * _LCAP < c[1]),
            round_body, (0, 0))

    table_pass(center_hbm, inT_hbm, center_out_hbm)
    table_pass(context_hbm, outT_hbm, context_out_hbm)


def kernel(center_indices, context_indices, in_embeddings, out_embeddings):
    center_wide, context_wide = _skipgram_stream(
        center_indices, context_indices, in_embeddings.T, out_embeddings.T)
    return (center_wide[:BATCH, :EMBED], context_wide[:BATCH, :EMBED])


# speedup vs baseline: 1.5669x; 1.5669x over previous
import functools

import jax
import jax.numpy as jnp
from jax import lax
from jax.experimental import pallas as pl
from jax.experimental.pallas import tpu as pltpu
from jax.experimental.pallas import tpu_sc as plsc

VOCAB = 1000000
EMBED = 64
BATCH = 16384

_INFO = plsc.get_sparse_core_info()
_NC = _INFO.num_cores
_NS = _INFO.num_subcores
_NW = _NC * _NS

_SLABW = 256
_NSLAB = VOCAB // _SLABW
_TAIL_BASE = _NSLAB * _SLABW
_TAILW = VOCAB - _TAIL_BASE
_SPW = _NSLAB // _NW
_SREM = _NSLAB % _NW
_LCAP = 1024
_STAGE = 128
_OUTROWS = BATCH + _NW
_IDXCH = 2048


@functools.partial(
    pl.kernel,
    mesh=plsc.VectorSubcoreMesh(core_axis_name="c", subcore_axis_name="s"),
    out_type=(
        jax.ShapeDtypeStruct((_OUTROWS, 2 * EMBED), jnp.float32),
        jax.ShapeDtypeStruct((_OUTROWS, 2 * EMBED), jnp.float32),
    ),
    scratch_types=[
        pltpu.VMEM((4, EMBED, _SLABW), jnp.float32),
        pltpu.VMEM((EMBED, _TAILW), jnp.float32),
        pltpu.VMEM((_IDXCH,), jnp.int32),
        pltpu.VMEM((_IDXCH,), jnp.int32),
        pltpu.VMEM((_LCAP + 16,), jnp.int32),
        pltpu.VMEM((_LCAP + 16,), jnp.int32),
        pltpu.VMEM((_LCAP + 16,), jnp.int32),
        pltpu.VMEM((_LCAP + 16,), jnp.int32),
        pltpu.VMEM((1, _STAGE), jnp.int32),
        pltpu.VMEM((_STAGE, 2 * EMBED), jnp.float32),
        pltpu.SemaphoreType.DMA,
        pltpu.SemaphoreType.DMA,
        pltpu.SemaphoreType.DMA,
        pltpu.SemaphoreType.DMA,
        pltpu.SemaphoreType.DMA,
        pltpu.SemaphoreType.DMA,
    ],
    compiler_params=pltpu.CompilerParams(needs_layout_passes=False),
)
def _skipgram_stream(center_hbm, context_hbm, inT_hbm, outT_hbm,
                     center_out_hbm, context_out_hbm,
                     slab_v, tail_v, idxa_v, idxb_v, mi_v, mk_v, scol_v, sk_v,
                     k2d_v, stage_v, sem0, sem1, sem2, sem3, isem0, isem1):
    wid = lax.axis_index("s") * _NC + lax.axis_index("c")
    lanes = lax.iota(jnp.int32, 16)
    zeros16 = lax.full((16,), 0, jnp.int32)
    sems = (sem0, sem1, sem2, sem3)
    isems = (isem0, isem1)
    ibufs = (idxa_v, idxb_v)

    s_lo = wid * _SPW + jnp.minimum(wid, _SREM)
    nslab = _SPW + jnp.where(wid < _SREM, 1, 0)
    lo = pl.multiple_of(s_lo * _SLABW, 128)
    is_last = wid == (_NW - 1)
    hi = jnp.where(is_last, VOCAB, lo + nslab * _SLABW)
    dummy_row = BATCH + wid

    def reset_k2d():
        row = k2d_v.at[0]
        for t in range(_STAGE // 16):
            row[pl.ds(t * 16, 16)] = zeros16 + dummy_row

    def flush(out_hbm):
        pltpu.sync_copy(stage_v, out_hbm.at[k2d_v.at[0]])
        reset_k2d()

    def fire(tblT_hbm, si, b):
        base = pl.multiple_of((s_lo + si) * _SLABW, 128)
        return pltpu.async_copy(
            tblT_hbm.at[:, pl.ds(base, _SLABW)], slab_v.at[b], sems[b])

    def process_slab(slabref, base, width, listn, st, out_hbm):
        def collect(v, sp):
            mi = mi_v[pl.ds(v * 16, 16)]
            kk = mk_v[pl.ds(v * 16, 16)]
            act = (v * 16 + lanes) < listn
            m = act & (mi >= base) & (mi < base + width)
            plsc.store_compressed(scol_v.at[pl.ds(sp, 16)], mi - base, mask=m)
            plsc.store_compressed(sk_v.at[pl.ds(sp, 16)], kk, mask=m)
            return sp + plsc.all_reduce_population_count(m)[0]

        sp = lax.fori_loop(0, (listn + 15) // 16, collect, 0)

        def extract(q, st):
            colv = scol_v[pl.ds(q * 16, 16)]
            kvv = sk_v[pl.ds(q * 16, 16)]
            mm = (q * 16 + lanes) < sp
            plsc.store_scatter(k2d_v.at[0], [st + lanes], kvv, mask=mm)

            def col_body(c, carry):
                val = plsc.load_gather(slabref, [zeros16 + c, colv], mask=mm)
                plsc.store_scatter(stage_v, [st + lanes, zeros16 + c], val,
                                  mask=mm)
                return carry

            lax.fori_loop(0, EMBED, col_body, 0)
            st = st + plsc.all_reduce_population_count(mm)[0]

            def do_flush():
                flush(out_hbm)
                return 0

            return lax.cond(st > _STAGE - 16, do_flush, lambda: st)

        return lax.fori_loop(0, (sp + 15) // 16, extract, st)

    def table_pass(idx_hbm, tblT_hbm, out_hbm):
        def round_body(carry):
            r, _ = carry
            reset_k2d()
            for b in range(4):
                fire(tblT_hbm, b, b)

            rbase = r * _LCAP

            def scan_chunk(ch_base, ibuf, pos):
                def scanb(b, pos):
                    iv = ibuf[pl.ds(b * 16, 16)]
                    kv = lanes + (ch_base + b * 16)
                    m = (iv >= lo) & (iv < hi)
                    cnt = plsc.all_reduce_population_count(m)[0]
                    lpos = pos - rbase

                    def fast_append():
                        at = jnp.maximum(lpos, 0)
                        plsc.store_compressed(mi_v.at[pl.ds(at, 16)], iv,
                                              mask=m)
                        plsc.store_compressed(mk_v.at[pl.ds(at, 16)], kv,
                                              mask=m)

                    def ranked_append():
                        rank = lpos + plsc.cumsum(
                            jnp.where(m, 1, 0).astype(jnp.int32)) - 1
                        acc = m & (rank >= 0) & (rank < _LCAP)
                        at = jnp.maximum(lpos, 0)
                        plsc.store_compressed(mi_v.at[pl.ds(at, 16)], iv,
                                              mask=acc)
                        plsc.store_compressed(mk_v.at[pl.ds(at, 16)], kv,
                                              mask=acc)

                    lax.cond((lpos >= 0) & (lpos <= _LCAP - 16),
                             fast_append, ranked_append)
                    return pos + cnt

                return lax.fori_loop(0, _IDXCH // 16, scanb, pos)

            pos = 0
            nch = BATCH // _IDXCH
            pltpu.async_copy(idx_hbm.at[pl.ds(0, _IDXCH)], ibufs[0],
                             isems[0])
            for ch in range(nch):
                if ch + 1 < nch:
                    pltpu.async_copy(
                        idx_hbm.at[pl.ds((ch + 1) * _IDXCH, _IDXCH)],
                        ibufs[(ch + 1) % 2], isems[(ch + 1) % 2])
                pltpu.make_async_copy(
                    idx_hbm.at[pl.ds(0, _IDXCH)], ibufs[ch % 2],
                    isems[ch % 2]).wait()
                pos = scan_chunk(ch * _IDXCH, ibufs[ch % 2], pos)
            total = pos
            listn = jnp.clip(total - rbase, 0, _LCAP)

            def pair_body(g, st):
                for b in range(4):
                    si = g * 4 + b

                    def do_slab(st=st, si=si, b=b):
                        pltpu.make_async_copy(
                            tblT_hbm.at[:, pl.ds(0, _SLABW)],
                            slab_v.at[b], sems[b]).wait()
                        base = lo + si * _SLABW
                        st2 = process_slab(slab_v.at[b], base, _SLABW,
                                           listn, st, out_hbm)

                        @pl.when(si + 4 < nslab)
                        def _():
                            fire(tblT_hbm, si + 4, b)

                        return st2

                    st = lax.cond(si < nslab, do_slab, lambda st=st: st)
                return st

            st = lax.fori_loop(0, (_SPW + 1 + 4) // 4, pair_body, 0)

            def do_tail(st=st):
                pltpu.sync_copy(
                    tblT_hbm.at[:, pl.ds(_TAIL_BASE, _TAILW)], tail_v)
                return process_slab(tail_v, _TAIL_BASE, _TAILW, listn, st,
                                    out_hbm)

            st = lax.cond(is_last, do_tail, lambda: st)

            @pl.when(st > 0)
            def _():
                flush(out_hbm)

            return r + 1, total

        lax.while_loop(
            lambda c: (c[0] == 0) | (c[0] * _LCAP < c[1]),
            round_body, (0, 0))

    table_pass(center_hbm, inT_hbm, center_out_hbm)
    table_pass(context_hbm, outT_hbm, context_out_hbm)


def kernel(center_indices, context_indices, in_embeddings, out_embeddings):
    center_wide, context_wide = _skipgram_stream(
        center_indices, context_indices, in_embeddings.T, out_embeddings.T)
    return (center_wide[:BATCH, :EMBED], context_wide[:BATCH, :EMBED])
